# Initial kernel scaffold; baseline (speedup 1.0000x reference)
import jax
import jax.numpy as jnp
from jax.experimental import pallas as pl


def kernel(x, codebook, training):
    raise NotImplementedError("write your pallas kernel here")


# speedup vs baseline: 1.6021x; 1.6021x over previous
import dataclasses
import functools
import math

import jax
import jax.numpy as jnp
from jax import lax
from jax.experimental import pallas as pl
from jax.experimental.pallas import tpu as pltpu
from jax.experimental.pallas import tpu_sc as plsc

NUM_CODE = 8192
CODE_DIM = 256
N_TOKENS = 8192

TOK_BLK = 256

_SC_CORES = 2
_SC_SUBCORES = 16
_SC_WORKERS = _SC_CORES * _SC_SUBCORES
_WIN = 128
_WINDOWS_PER_WORKER = N_TOKENS // (_SC_WORKERS * _WIN)


def _argmin_body(xx_ref, x_ref, cb_ref, yy_ref, idx_ref):
    x = x_ref[...]
    cb = cb_ref[...]
    xy = lax.dot_general(
        x, cb, (((1,), (1,)), ((), ())),
        preferred_element_type=jnp.float32,
    )
    d = (xx_ref[...] - 2.0 * xy) + yy_ref[...]
    rowmin = jnp.min(d, axis=1, keepdims=True)
    iota = lax.broadcasted_iota(jnp.int32, d.shape, 1)
    idx = jnp.min(
        jnp.where(d == rowmin, iota, jnp.int32(NUM_CODE)),
        axis=1, keepdims=True,
    )
    idx_ref[...] = idx


def _compute_code_idx(x, codebook, xx, yy):
    grid = (N_TOKENS // TOK_BLK,)
    return pl.pallas_call(
        _argmin_body,
        grid=grid,
        in_specs=[
            pl.BlockSpec((TOK_BLK, 1), lambda i: (i, 0)),
            pl.BlockSpec((TOK_BLK, CODE_DIM), lambda i: (i, 0)),
            pl.BlockSpec((NUM_CODE, CODE_DIM), lambda i: (0, 0)),
            pl.BlockSpec((1, NUM_CODE), lambda i: (0, 0)),
        ],
        out_specs=pl.BlockSpec((TOK_BLK, 1), lambda i: (i, 0)),
        out_shape=jax.ShapeDtypeStruct((N_TOKENS, 1), jnp.int32),
    )(xx, x, codebook, yy)


def _sc_gather_hist(code_idx, codebook):
    mesh = plsc.VectorSubcoreMesh(core_axis_name="c", subcore_axis_name="s")
    cp = pltpu.CompilerParams()
    if "needs_layout_passes" in pltpu.CompilerParams.__dataclass_fields__:
        cp = dataclasses.replace(cp, needs_layout_passes=False)

    @functools.partial(
        pl.kernel,
        mesh=mesh,
        compiler_params=cp,
        out_type=(
            jax.ShapeDtypeStruct((N_TOKENS, CODE_DIM), jnp.float32),
            jax.ShapeDtypeStruct((_SC_WORKERS, NUM_CODE), jnp.float32),
        ),
        scratch_types=[
            pltpu.VMEM((_WIN,), jnp.int32),
            pltpu.VMEM((_WIN, CODE_DIM), jnp.float32),
            pltpu.VMEM((NUM_CODE,), jnp.float32),
            pltpu.SemaphoreType.DMA,
        ],
    )
    def k(idx_hbm, cb_hbm, xd_hbm, hist_hbm, idx_v, rows_v, hist_v, sem):
        wid = lax.axis_index("s") * _SC_CORES + lax.axis_index("c")

        @pl.loop(0, NUM_CODE, step=16)
        def _(i):
            hist_v[pl.ds(i, 16)] = jnp.zeros((16,), jnp.float32)

        ones = jnp.full((16,), 1.0, jnp.float32)
        for t in range(_WINDOWS_PER_WORKER):
            base = (wid * _WINDOWS_PER_WORKER + t) * _WIN
            pltpu.sync_copy(idx_hbm.at[pl.ds(base, _WIN)], idx_v)
            pltpu.async_copy(cb_hbm.at[idx_v], rows_v, sem).wait()
            pltpu.sync_copy(rows_v, xd_hbm.at[pl.ds(base, _WIN)])
            for j in range(_WIN // 16):
                iv = idx_v[pl.ds(j * 16, 16)]
                plsc.addupdate_scatter(hist_v, [iv], ones)

        pltpu.sync_copy(hist_v, hist_hbm.at[wid])

    return k(code_idx, codebook)


def _plx_body(h_ref, out_ref):
    c = jnp.sum(h_ref[...], axis=0, keepdims=True)
    total = jnp.sum(c)
    prob = c / jnp.maximum(total, 1e-8)
    plx = jnp.exp(-jnp.sum(prob * jnp.log(prob + 1e-7)))
    out_ref[...] = jnp.full((1, 1), plx, jnp.float32)


def _compute_perplexity(hist):
    out = pl.pallas_call(
        _plx_body,
        out_shape=jax.ShapeDtypeStruct((1, 1), jnp.float32),
    )(hist)
    return out.reshape(())


def kernel(x, codebook, training):
    xx = jnp.sum(x ** 2, axis=-1, keepdims=True)
    k_w = codebook.T
    yy = jnp.sum(k_w ** 2, axis=0, keepdims=True)
    code_idx = _compute_code_idx(x, codebook, xx, yy)
    x_d, hist = _sc_gather_hist(code_idx.reshape(N_TOKENS), codebook)
    perplexity = _compute_perplexity(hist)
    return (x_d, perplexity)


# speedup vs baseline: 1.6635x; 1.0383x over previous
import dataclasses
import functools
import math

import jax
import jax.numpy as jnp
from jax import lax
from jax.experimental import pallas as pl
from jax.experimental.pallas import tpu as pltpu
from jax.experimental.pallas import tpu_sc as plsc

NUM_CODE = 8192
CODE_DIM = 256
N_TOKENS = 8192

TOK_BLK = 256

_SC_CORES = 2
_SC_SUBCORES = 16
_SC_WORKERS = _SC_CORES * _SC_SUBCORES
_WIN = 128
_WINDOWS_PER_WORKER = N_TOKENS // (_SC_WORKERS * _WIN)


def _argmin_body(xx_ref, x2_ref, cb_ref, yy_ref, it_ref, idx_ref):
    x2 = x2_ref[...]
    cb = cb_ref[...]
    xy2 = lax.dot_general(
        x2, cb, (((1,), (1,)), ((), ())),
        preferred_element_type=jnp.float32,
    )
    d = (xx_ref[...] - xy2) + yy_ref[...]
    rowmin = jnp.min(d, axis=1, keepdims=True)
    idx_f = jnp.min(
        jnp.where(d == rowmin, it_ref[...], jnp.float32(NUM_CODE)),
        axis=1, keepdims=True,
    )
    idx_ref[...] = idx_f.astype(jnp.int32)


def _compute_code_idx(x, codebook, xx, yy):
    grid = (N_TOKENS // TOK_BLK,)
    x2 = x + x
    iota = lax.broadcasted_iota(jnp.float32, (1, NUM_CODE), 1)
    return pl.pallas_call(
        _argmin_body,
        grid=grid,
        in_specs=[
            pl.BlockSpec((TOK_BLK, 1), lambda i: (i, 0)),
            pl.BlockSpec((TOK_BLK, CODE_DIM), lambda i: (i, 0)),
            pl.BlockSpec((NUM_CODE, CODE_DIM), lambda i: (0, 0)),
            pl.BlockSpec((1, NUM_CODE), lambda i: (0, 0)),
            pl.BlockSpec((1, NUM_CODE), lambda i: (0, 0)),
        ],
        out_specs=pl.BlockSpec((TOK_BLK, 1), lambda i: (i, 0)),
        out_shape=jax.ShapeDtypeStruct((N_TOKENS, 1), jnp.int32),
    )(xx, x2, codebook, yy, iota)


def _sc_gather_hist(code_idx, codebook):
    mesh = plsc.VectorSubcoreMesh(core_axis_name="c", subcore_axis_name="s")
    cp = pltpu.CompilerParams()
    if "needs_layout_passes" in pltpu.CompilerParams.__dataclass_fields__:
        cp = dataclasses.replace(cp, needs_layout_passes=False)

    @functools.partial(
        pl.kernel,
        mesh=mesh,
        compiler_params=cp,
        out_type=(
            jax.ShapeDtypeStruct((N_TOKENS, CODE_DIM), jnp.float32),
            jax.ShapeDtypeStruct((_SC_WORKERS, NUM_CODE), jnp.float32),
        ),
        scratch_types=[
            pltpu.VMEM((_WIN,), jnp.int32),
            pltpu.VMEM((_WIN, CODE_DIM), jnp.float32),
            pltpu.VMEM((NUM_CODE,), jnp.float32),
            pltpu.SemaphoreType.DMA,
        ],
    )
    def k(idx_hbm, cb_hbm, xd_hbm, hist_hbm, idx_v, rows_v, hist_v, sem):
        wid = lax.axis_index("s") * _SC_CORES + lax.axis_index("c")

        @pl.loop(0, NUM_CODE, step=16)
        def _(i):
            hist_v[pl.ds(i, 16)] = jnp.zeros((16,), jnp.float32)

        ones = jnp.full((16,), 1.0, jnp.float32)
        for t in range(_WINDOWS_PER_WORKER):
            base = (wid * _WINDOWS_PER_WORKER + t) * _WIN
            pltpu.sync_copy(idx_hbm.at[pl.ds(base, _WIN)], idx_v)
            pltpu.async_copy(cb_hbm.at[idx_v], rows_v, sem).wait()
            pltpu.sync_copy(rows_v, xd_hbm.at[pl.ds(base, _WIN)])
            for j in range(_WIN // 16):
                iv = idx_v[pl.ds(j * 16, 16)]
                plsc.addupdate_scatter(hist_v, [iv], ones)

        pltpu.sync_copy(hist_v, hist_hbm.at[wid])

    return k(code_idx, codebook)


def _plx_body(h_ref, out_ref):
    c = jnp.sum(h_ref[...], axis=0, keepdims=True)
    total = jnp.sum(c)
    prob = c / jnp.maximum(total, 1e-8)
    plx = jnp.exp(-jnp.sum(prob * jnp.log(prob + 1e-7)))
    out_ref[...] = jnp.full((1, 1), plx, jnp.float32)


def _compute_perplexity(hist):
    out = pl.pallas_call(
        _plx_body,
        out_shape=jax.ShapeDtypeStruct((1, 1), jnp.float32),
    )(hist)
    return out.reshape(())


def kernel(x, codebook, training):
    xx = jnp.sum(x ** 2, axis=-1, keepdims=True)
    k_w = codebook.T
    yy = jnp.sum(k_w ** 2, axis=0, keepdims=True)
    code_idx = _compute_code_idx(x, codebook, xx, yy)
    x_d, hist = _sc_gather_hist(code_idx.reshape(N_TOKENS), codebook)
    perplexity = _compute_perplexity(hist)
    return (x_d, perplexity)


# speedup vs baseline: 1.8262x; 1.0978x over previous
import dataclasses
import functools
import math

import jax
import jax.numpy as jnp
from jax import lax
from jax.experimental import pallas as pl
from jax.experimental.pallas import tpu as pltpu
from jax.experimental.pallas import tpu_sc as plsc

NUM_CODE = 8192
CODE_DIM = 256
N_TOKENS = 8192

TOK_BLK = 1024

_SC_CORES = 2
_SC_SUBCORES = 16
_SC_WORKERS = _SC_CORES * _SC_SUBCORES
_WIN = 128
_WINDOWS_PER_WORKER = N_TOKENS // (_SC_WORKERS * _WIN)


def _argmin_body(xx_ref, x_ref, cb_ref, yy_ref, it_ref, idx_ref):
    xv = x_ref[...]
    x2 = xv + xv
    cb = cb_ref[...]
    xy2 = lax.dot_general(
        x2, cb, (((1,), (1,)), ((), ())),
        preferred_element_type=jnp.float32,
    )
    d = (xx_ref[...] - xy2) + yy_ref[...]
    rowmin = jnp.min(d, axis=1, keepdims=True)
    idx_f = jnp.min(
        jnp.where(d == rowmin, it_ref[...], jnp.float32(NUM_CODE)),
        axis=1, keepdims=True,
    )
    idx_ref[...] = idx_f.astype(jnp.int32)


def _compute_code_idx(x, codebook, xx, yy):
    grid = (N_TOKENS // TOK_BLK,)
    iota = lax.broadcasted_iota(jnp.float32, (1, NUM_CODE), 1)
    return pl.pallas_call(
        _argmin_body,
        grid=grid,
        in_specs=[
            pl.BlockSpec((TOK_BLK, 1), lambda i: (i, 0)),
            pl.BlockSpec((TOK_BLK, CODE_DIM), lambda i: (i, 0)),
            pl.BlockSpec((NUM_CODE, CODE_DIM), lambda i: (0, 0)),
            pl.BlockSpec((1, NUM_CODE), lambda i: (0, 0)),
            pl.BlockSpec((1, NUM_CODE), lambda i: (0, 0)),
        ],
        out_specs=pl.BlockSpec((TOK_BLK, 1), lambda i: (i, 0)),
        out_shape=jax.ShapeDtypeStruct((N_TOKENS, 1), jnp.int32),
    )(xx, x, codebook, yy, iota)


def _sc_gather_hist(code_idx, codebook):
    mesh = plsc.VectorSubcoreMesh(core_axis_name="c", subcore_axis_name="s")
    cp = pltpu.CompilerParams()
    if "needs_layout_passes" in pltpu.CompilerParams.__dataclass_fields__:
        cp = dataclasses.replace(cp, needs_layout_passes=False)

    @functools.partial(
        pl.kernel,
        mesh=mesh,
        compiler_params=cp,
        out_type=(
            jax.ShapeDtypeStruct((N_TOKENS, CODE_DIM), jnp.float32),
            jax.ShapeDtypeStruct((_SC_WORKERS, NUM_CODE), jnp.float32),
        ),
        scratch_types=[
            pltpu.VMEM((_WIN,), jnp.int32),
            pltpu.VMEM((_WIN, CODE_DIM), jnp.float32),
            pltpu.VMEM((NUM_CODE,), jnp.float32),
            pltpu.SemaphoreType.DMA,
        ],
    )
    def k(idx_hbm, cb_hbm, xd_hbm, hist_hbm, idx_v, rows_v, hist_v, sem):
        wid = lax.axis_index("s") * _SC_CORES + lax.axis_index("c")

        @pl.loop(0, NUM_CODE, step=16)
        def _(i):
            hist_v[pl.ds(i, 16)] = jnp.zeros((16,), jnp.float32)

        ones = jnp.full((16,), 1.0, jnp.float32)
        for t in range(_WINDOWS_PER_WORKER):
            base = (wid * _WINDOWS_PER_WORKER + t) * _WIN
            pltpu.sync_copy(idx_hbm.at[pl.ds(base, _WIN)], idx_v)
            pltpu.async_copy(cb_hbm.at[idx_v], rows_v, sem).wait()
            pltpu.sync_copy(rows_v, xd_hbm.at[pl.ds(base, _WIN)])
            for j in range(_WIN // 16):
                iv = idx_v[pl.ds(j * 16, 16)]
                plsc.addupdate_scatter(hist_v, [iv], ones)

        pltpu.sync_copy(hist_v, hist_hbm.at[wid])

    return k(code_idx, codebook)


def _plx_body(h_ref, out_ref):
    c = jnp.sum(h_ref[...], axis=0, keepdims=True)
    total = jnp.sum(c)
    prob = c / jnp.maximum(total, 1e-8)
    plx = jnp.exp(-jnp.sum(prob * jnp.log(prob + 1e-7)))
    out_ref[...] = jnp.full((1, 1), plx, jnp.float32)


def _compute_perplexity(hist):
    out = pl.pallas_call(
        _plx_body,
        out_shape=jax.ShapeDtypeStruct((1, 1), jnp.float32),
    )(hist)
    return out.reshape(())


def kernel(x, codebook, training):
    xx = jnp.sum(x ** 2, axis=-1, keepdims=True)
    k_w = codebook.T
    yy = jnp.sum(k_w ** 2, axis=0, keepdims=True)
    code_idx = _compute_code_idx(x, codebook, xx, yy)
    x_d, hist = _sc_gather_hist(code_idx.reshape(N_TOKENS), codebook)
    perplexity = _compute_perplexity(hist)
    return (x_d, perplexity)


# speedup vs baseline: 2.0265x; 1.1097x over previous
import dataclasses
import functools
import math

import jax
import jax.numpy as jnp
from jax import lax
from jax.experimental import pallas as pl
from jax.experimental.pallas import tpu as pltpu
from jax.experimental.pallas import tpu_sc as plsc

NUM_CODE = 8192
CODE_DIM = 256
N_TOKENS = 8192

TOK_BLK = 1024

_SC_CORES = 2
_SC_SUBCORES = 16
_SC_WORKERS = _SC_CORES * _SC_SUBCORES
_WIN = 128
_WINDOWS_PER_WORKER = N_TOKENS // (_SC_WORKERS * _WIN)


def _argmin_body(xx_ref, x_ref, cb_ref, yy_ref, idx_ref):
    xv = x_ref[...]
    x2 = xv + xv
    cb = cb_ref[...]
    xy2 = lax.dot_general(
        x2, cb, (((1,), (1,)), ((), ())),
        preferred_element_type=jnp.float32,
    )
    d = (xx_ref[...] - xy2) + yy_ref[...]
    idx_ref[...] = jnp.argmin(d, axis=1).reshape(-1, 1).astype(jnp.int32)


def _compute_code_idx(x, codebook, xx, yy):
    grid = (N_TOKENS // TOK_BLK,)
    return pl.pallas_call(
        _argmin_body,
        grid=grid,
        in_specs=[
            pl.BlockSpec((TOK_BLK, 1), lambda i: (i, 0)),
            pl.BlockSpec((TOK_BLK, CODE_DIM), lambda i: (i, 0)),
            pl.BlockSpec((NUM_CODE, CODE_DIM), lambda i: (0, 0)),
            pl.BlockSpec((1, NUM_CODE), lambda i: (0, 0)),
        ],
        out_specs=pl.BlockSpec((TOK_BLK, 1), lambda i: (i, 0)),
        out_shape=jax.ShapeDtypeStruct((N_TOKENS, 1), jnp.int32),
    )(xx, x, codebook, yy)


def _sc_gather_hist(code_idx, codebook):
    mesh = plsc.VectorSubcoreMesh(core_axis_name="c", subcore_axis_name="s")
    cp = pltpu.CompilerParams()
    if "needs_layout_passes" in pltpu.CompilerParams.__dataclass_fields__:
        cp = dataclasses.replace(cp, needs_layout_passes=False)

    @functools.partial(
        pl.kernel,
        mesh=mesh,
        compiler_params=cp,
        out_type=(
            jax.ShapeDtypeStruct((N_TOKENS, CODE_DIM), jnp.float32),
            jax.ShapeDtypeStruct((_SC_WORKERS, NUM_CODE), jnp.float32),
        ),
        scratch_types=[
            pltpu.VMEM((_WIN,), jnp.int32),
            pltpu.VMEM((_WIN,), jnp.int32),
            pltpu.VMEM((_WIN, CODE_DIM), jnp.float32),
            pltpu.VMEM((_WIN, CODE_DIM), jnp.float32),
            pltpu.VMEM((NUM_CODE,), jnp.float32),
            pltpu.SemaphoreType.DMA,
            pltpu.SemaphoreType.DMA,
            pltpu.SemaphoreType.DMA,
        ],
    )
    def k(idx_hbm, cb_hbm, xd_hbm, hist_hbm,
          idx0_v, idx1_v, rows0_v, rows1_v, hist_v, sem0, sem1, sem2):
        wid = lax.axis_index("s") * _SC_CORES + lax.axis_index("c")
        base0 = wid * (2 * _WIN)
        base1 = base0 + _WIN

        @pl.loop(0, NUM_CODE, step=16)
        def _(i):
            hist_v[pl.ds(i, 16)] = jnp.zeros((16,), jnp.float32)

        pltpu.sync_copy(idx_hbm.at[pl.ds(base0, _WIN)], idx0_v)
        pltpu.sync_copy(idx_hbm.at[pl.ds(base1, _WIN)], idx1_v)
        g0 = pltpu.async_copy(cb_hbm.at[idx0_v], rows0_v, sem0)
        g1 = pltpu.async_copy(cb_hbm.at[idx1_v], rows1_v, sem1)

        ones = jnp.full((16,), 1.0, jnp.float32)
        for j in range(_WIN // 16):
            plsc.addupdate_scatter(hist_v, [idx0_v[pl.ds(j * 16, 16)]], ones)
        for j in range(_WIN // 16):
            plsc.addupdate_scatter(hist_v, [idx1_v[pl.ds(j * 16, 16)]], ones)

        g0.wait()
        w0 = pltpu.async_copy(rows0_v, xd_hbm.at[pl.ds(base0, _WIN)], sem2)
        g1.wait()
        pltpu.sync_copy(rows1_v, xd_hbm.at[pl.ds(base1, _WIN)])
        w0.wait()
        pltpu.sync_copy(hist_v, hist_hbm.at[wid])

    return k(code_idx, codebook)


def _plx_body(h_ref, out_ref):
    c = jnp.sum(h_ref[...], axis=0, keepdims=True)
    total = jnp.sum(c)
    prob = c / jnp.maximum(total, 1e-8)
    plx = jnp.exp(-jnp.sum(prob * jnp.log(prob + 1e-7)))
    out_ref[...] = jnp.full((1, 1), plx, jnp.float32)


def _compute_perplexity(hist):
    out = pl.pallas_call(
        _plx_body,
        out_shape=jax.ShapeDtypeStruct((1, 1), jnp.float32),
    )(hist)
    return out.reshape(())


def kernel(x, codebook, training):
    xx = jnp.sum(x ** 2, axis=-1, keepdims=True)
    k_w = codebook.T
    yy = jnp.sum(k_w ** 2, axis=0, keepdims=True)
    code_idx = _compute_code_idx(x, codebook, xx, yy)
    x_d, hist = _sc_gather_hist(code_idx.reshape(N_TOKENS), codebook)
    perplexity = _compute_perplexity(hist)
    return (x_d, perplexity)
